# baseline (speedup 1.0000x reference)
import functools

import jax
import jax.numpy as jnp
from jax import lax
from jax.experimental import pallas as pl
from jax.experimental.pallas import tpu as pltpu
from jax.experimental.pallas import tpu_sc as plsc

B, L, H, V = 4096, 200, 64, 1000000
N = B * L

_info = plsc.get_sparse_core_info()
NC, NS, LANES = _info.num_cores, _info.num_subcores, _info.num_lanes
NW = NC * NS
S = N // NW
R = 400
K = R // L
C = S // R
RG = 100
G = R // RG
HV = H // LANES


def _sc_embed(x_hbm, table_hbm, pos_hbm, out_hbm, idx_v, rows_v, pos_v, sem):
    wid = lax.axis_index("s") * NC + lax.axis_index("c")
    base = wid * (S // RG)

    pltpu.sync_copy(x_hbm.at[pl.ds(base, S // RG)], idx_v)
    for rep in range(K):
        pltpu.sync_copy(pos_hbm, pos_v.at[pl.ds(rep * L, L)])

    def chunk_body(c, carry):
        cps = [
            pltpu.async_copy(
                table_hbm.at[idx_v.at[c * G + g]],
                rows_v.at[pl.ds(g * RG, RG)],
                sem,
            )
            for g in range(G)
        ]
        for cp in cps:
            cp.wait()

        def row_body(j, carry2):
            for cc in range(HV):
                sl = pl.ds(cc * LANES, LANES)
                rows_v[j, sl] = rows_v[j, sl] + pos_v[j, sl]
            return carry2

        lax.fori_loop(0, R, row_body, 0)

        pltpu.sync_copy(rows_v, out_hbm.at[pl.ds(wid * S + c * R, R)])
        return carry

    lax.fori_loop(0, C, chunk_body, 0)


@functools.partial(jax.jit, static_argnums=())
def kernel(x, embed_table, pos_table):
    xf = x.reshape(N // RG, RG)
    mesh = plsc.VectorSubcoreMesh(core_axis_name="c", subcore_axis_name="s")
    run = pl.kernel(
        _sc_embed,
        out_type=jax.ShapeDtypeStruct((N, H), jnp.float32),
        mesh=mesh,
        compiler_params=pltpu.CompilerParams(use_tc_tiling_on_sc=False),
        scratch_types=[
            pltpu.VMEM((S // RG, RG), jnp.int32),
            pltpu.VMEM((R, H), jnp.float32),
            pltpu.VMEM((R, H), jnp.float32),
            pltpu.SemaphoreType.DMA,
        ],
    )
    out = run(xf, embed_table, pos_table)
    return out.reshape(B, L, H)
